# baseline (speedup 1.0000x reference)
import jax
import jax.numpy as jnp
from jax.experimental import pallas as pl

_DIM = 4096
_E = 64
_K = 8
_TBLK = 512


def _router_block(x_ref, wt_ref, ew_ref, ei_ref, probs_ref, usage_ref,
                  lb_ref, z_ref):
    i = pl.program_id(0)
    nblk = pl.num_programs(0)
    n_tokens = nblk * _TBLK

    logits = jnp.dot(x_ref[...], wt_ref[...],
                     preferred_element_type=jnp.float32)

    m = jnp.max(logits, axis=-1, keepdims=True)
    ex = jnp.exp(logits - m)
    s = jnp.sum(ex, axis=-1, keepdims=True)
    probs = ex / s
    probs_ref[...] = probs

    lse = m + jnp.log(s)
    z_part = jnp.sum(lse * lse)

    usage_part = jnp.sum(probs, axis=0, keepdims=True)

    lanes = jax.lax.broadcasted_iota(jnp.int32, logits.shape, 1)
    work = logits
    vals, idxs = [], []
    for _ in range(_K):
        mk = jnp.max(work, axis=-1, keepdims=True)
        ik = jnp.min(jnp.where(work == mk, lanes, _E), axis=-1,
                     keepdims=True)
        vals.append(mk)
        idxs.append(ik)
        work = jnp.where(lanes == ik, -jnp.inf, work)
    topv = jnp.concatenate(vals, axis=-1)
    topi = jnp.concatenate(idxs, axis=-1)

    et = jnp.exp(topv - topv[:, :1])
    ew_ref[...] = et / jnp.sum(et, axis=-1, keepdims=True)
    ei_ref[...] = topi

    @pl.when(i == 0)
    def _init():
        usage_ref[...] = jnp.zeros_like(usage_ref)
        z_ref[...] = jnp.zeros_like(z_ref)
        lb_ref[...] = jnp.zeros_like(lb_ref)

    usage_ref[...] += usage_part
    z_ref[...] += jnp.full((1, 1), z_part, jnp.float32)

    @pl.when(i == nblk - 1)
    def _finalize():
        usage = usage_ref[...] / n_tokens
        usage_ref[...] = usage
        lb_ref[...] = jnp.full((1, 1), jnp.sum(usage * usage) * _E,
                               jnp.float32)
        z_ref[...] = z_ref[...] / n_tokens


def kernel(x, W):
    b, s, d = x.shape
    n = b * s
    x2 = x.reshape(n, d)
    wt = W.T
    grid = n // _TBLK

    ew, ei, probs, usage, lb, z = pl.pallas_call(
        _router_block,
        grid=(grid,),
        in_specs=[
            pl.BlockSpec((_TBLK, _DIM), lambda i: (i, 0)),
            pl.BlockSpec((_DIM, _E), lambda i: (0, 0)),
        ],
        out_specs=[
            pl.BlockSpec((_TBLK, _K), lambda i: (i, 0)),
            pl.BlockSpec((_TBLK, _K), lambda i: (i, 0)),
            pl.BlockSpec((_TBLK, _E), lambda i: (i, 0)),
            pl.BlockSpec((1, _E), lambda i: (0, 0)),
            pl.BlockSpec((1, 1), lambda i: (0, 0)),
            pl.BlockSpec((1, 1), lambda i: (0, 0)),
        ],
        out_shape=[
            jax.ShapeDtypeStruct((n, _K), jnp.float32),
            jax.ShapeDtypeStruct((n, _K), jnp.int32),
            jax.ShapeDtypeStruct((n, _E), jnp.float32),
            jax.ShapeDtypeStruct((1, _E), jnp.float32),
            jax.ShapeDtypeStruct((1, 1), jnp.float32),
            jax.ShapeDtypeStruct((1, 1), jnp.float32),
        ],
    )(x2, wt)

    return (ew.reshape(b, s, _K), ei.reshape(b, s, _K),
            lb.reshape(()), z.reshape(()), usage.reshape(_E),
            probs.reshape(b, s, _E))


# speedup vs baseline: 1.1931x; 1.1931x over previous
import jax
import jax.numpy as jnp
from jax.experimental import pallas as pl

_DIM = 4096
_E = 64
_K = 8
_TBLK = 256


def _router_block(x_ref, w_ref, ew_ref, ei_ref, probs_ref, usage_ref,
                  lb_ref, z_ref):
    i = pl.program_id(0)
    nblk = pl.num_programs(0)
    n_tokens = nblk * _TBLK

    logits_t = jax.lax.dot_general(
        w_ref[...], x_ref[...], (((1,), (1,)), ((), ())),
        preferred_element_type=jnp.float32)

    rows = jax.lax.broadcasted_iota(jnp.int32, logits_t.shape, 0)
    work = logits_t
    vals, idxs = [], []
    for _ in range(_K):
        mk = jnp.max(work, axis=0, keepdims=True)
        ik = jnp.min(jnp.where(work == mk, rows, _E), axis=0,
                     keepdims=True)
        vals.append(mk)
        idxs.append(ik)
        work = jnp.where(rows == ik, -jnp.inf, work)
    topv = jnp.concatenate(vals, axis=0)
    topi = jnp.concatenate(idxs, axis=0)

    m = topv[0:1, :]
    ex = jnp.exp(logits_t - m)
    s = jnp.sum(ex, axis=0, keepdims=True)
    probs_t = ex / s
    probs_ref[...] = probs_t.T

    lse = m + jnp.log(s)
    z_part = jnp.sum(lse * lse)
    usage_part = jnp.sum(probs_t, axis=1, keepdims=True)

    et = jnp.exp(topv - m)
    ew_ref[...] = (et / jnp.sum(et, axis=0, keepdims=True)).T
    ei_ref[...] = topi.T

    @pl.when(i == 0)
    def _init():
        usage_ref[...] = jnp.zeros_like(usage_ref)
        z_ref[...] = jnp.zeros_like(z_ref)
        lb_ref[...] = jnp.zeros_like(lb_ref)

    usage_ref[...] += usage_part
    z_ref[...] += jnp.full((1, 1), z_part, jnp.float32)

    @pl.when(i == nblk - 1)
    def _finalize():
        usage = usage_ref[...] / n_tokens
        usage_ref[...] = usage
        lb_ref[...] = jnp.full((1, 1), jnp.sum(usage * usage) * _E,
                               jnp.float32)
        z_ref[...] = z_ref[...] / n_tokens


def kernel(x, W):
    b, s, d = x.shape
    n = b * s
    x2 = x.reshape(n, d)
    grid = n // _TBLK

    ew, ei, probs, usage, lb, z = pl.pallas_call(
        _router_block,
        grid=(grid,),
        in_specs=[
            pl.BlockSpec((_TBLK, _DIM), lambda i: (i, 0)),
            pl.BlockSpec((_E, _DIM), lambda i: (0, 0)),
        ],
        out_specs=[
            pl.BlockSpec((_TBLK, _K), lambda i: (i, 0)),
            pl.BlockSpec((_TBLK, _K), lambda i: (i, 0)),
            pl.BlockSpec((_TBLK, _E), lambda i: (i, 0)),
            pl.BlockSpec((_E, 1), lambda i: (0, 0)),
            pl.BlockSpec((1, 1), lambda i: (0, 0)),
            pl.BlockSpec((1, 1), lambda i: (0, 0)),
        ],
        out_shape=[
            jax.ShapeDtypeStruct((n, _K), jnp.float32),
            jax.ShapeDtypeStruct((n, _K), jnp.int32),
            jax.ShapeDtypeStruct((n, _E), jnp.float32),
            jax.ShapeDtypeStruct((_E, 1), jnp.float32),
            jax.ShapeDtypeStruct((1, 1), jnp.float32),
            jax.ShapeDtypeStruct((1, 1), jnp.float32),
        ],
    )(x2, W)

    return (ew.reshape(b, s, _K), ei.reshape(b, s, _K),
            lb.reshape(()), z.reshape(()), usage.reshape(_E),
            probs.reshape(b, s, _E))
